# baseline (speedup 1.0000x reference)
import functools

import jax
import jax.numpy as jnp
from jax import lax
from jax.experimental import pallas as pl
from jax.experimental.pallas import tpu as pltpu
from jax.experimental.pallas import tpu_sc as plsc

_NC = 2
_NS = 16
_NW = _NC * _NS


def _router_body(logits_ref, dest_ref, scale_ref, aux_ref, kc_ref, *, T, E, K, CAP, TRASH):
    logits = logits_ref[...]
    iota_e = jax.lax.broadcasted_iota(jnp.int32, (T, E), 1)
    m1 = jnp.max(logits, axis=1, keepdims=True)
    a1 = jnp.min(jnp.where(logits == m1, iota_e, E), axis=1, keepdims=True)
    neg = jnp.float32(-jnp.inf)
    masked = jnp.where(iota_e == a1, neg, logits)
    m2 = jnp.max(masked, axis=1, keepdims=True)
    a2 = jnp.min(jnp.where(masked == m2, iota_e, E), axis=1, keepdims=True)
    z = jnp.exp(m2 - m1)
    g1 = 1.0 / (1.0 + z)
    g2 = z / (1.0 + z)
    p = jnp.exp(logits - m1)
    probs = p / jnp.sum(p, axis=1, keepdims=True)
    imp = jnp.mean(probs, axis=0, keepdims=True)
    oh1 = (iota_e == a1).astype(jnp.float32)
    oh2 = (iota_e == a2).astype(jnp.float32)
    c = oh1 + oh2
    counts = jnp.sum(c, axis=0, keepdims=True)
    aux_ref[...] = jnp.reshape(E * jnp.sum(imp * counts) / (T * K), (1, 1))
    kc_ref[...] = jnp.minimum(counts, CAP).astype(jnp.int32)
    s = c
    sh = 1
    while sh < T:
        s = s + jnp.concatenate(
            [jnp.zeros((sh, E), jnp.float32), s[: T - sh]], axis=0)
        sh *= 2
    cex = s - c
    pos1 = jnp.sum(cex * oh1, axis=1, keepdims=True).astype(jnp.int32)
    pos2 = jnp.sum(cex * oh2, axis=1, keepdims=True).astype(jnp.int32)
    keep1 = pos1 < CAP
    keep2 = pos2 < CAP
    d1 = jnp.where(keep1, a1 * CAP + jnp.minimum(pos1, CAP - 1), TRASH)
    d2 = jnp.where(keep2, a2 * CAP + jnp.minimum(pos2, CAP - 1), TRASH)
    dest_ref[...] = jnp.concatenate([d1, d2], axis=1)
    scale_ref[...] = jnp.concatenate(
        [jnp.where(keep1, g1, 0.0), jnp.where(keep2, g2, 0.0)], axis=1)


def _dispatch_body(hs_hbm, d0_hbm, d1_hbm, buf_hbm, rows_v, idx0_v, idx1_v, sem,
                   *, tpw, ch):
    wid = lax.axis_index("s") * _NC + lax.axis_index("c")
    base = wid * tpw
    pltpu.sync_copy(d0_hbm.at[wid], idx0_v)
    pltpu.sync_copy(d1_hbm.at[wid], idx1_v)
    for j in range(tpw // ch):
        pltpu.sync_copy(hs_hbm.at[pl.ds(base + j * ch, ch)], rows_v)
        pltpu.async_copy(rows_v, buf_hbm.at[idx0_v.at[j]], sem).wait()
        pltpu.async_copy(rows_v, buf_hbm.at[idx1_v.at[j]], sem).wait()


def _combine_body(y_hbm, d0_hbm, d1_hbm, g0_hbm, g1_hbm, rows_v, idx0_v, idx1_v,
                  sem, *, tpw, ch):
    wid = lax.axis_index("s") * _NC + lax.axis_index("c")
    base = wid * tpw
    pltpu.sync_copy(d0_hbm.at[wid], idx0_v)
    pltpu.sync_copy(d1_hbm.at[wid], idx1_v)
    for j in range(tpw // ch):
        pltpu.async_copy(y_hbm.at[idx0_v.at[j]], rows_v, sem).wait()
        pltpu.sync_copy(rows_v, g0_hbm.at[pl.ds(base + j * ch, ch)])
        pltpu.async_copy(y_hbm.at[idx1_v.at[j]], rows_v, sem).wait()
        pltpu.sync_copy(rows_v, g1_hbm.at[pl.ds(base + j * ch, ch)])


def _ffn_body(kc_ref, buf_ref, w1_ref, w2_ref, y_ref, *, rbf, cap):
    e = pl.program_id(0)
    f = pl.program_id(1)
    kc = kc_ref[e]
    w1b = w1_ref[0].astype(jnp.bfloat16)
    w2b = w2_ref[0].astype(jnp.bfloat16)
    for rb in range(cap // rbf):
        @pl.when(rb * rbf < kc)
        def _(rb=rb):
            sl = pl.ds(rb * rbf, rbf)
            x = buf_ref[sl, :].astype(jnp.bfloat16)
            h = jax.nn.gelu(jax.lax.dot_general(
                x, w1b, (((1,), (0,)), ((), ())),
                preferred_element_type=jnp.float32))
            contrib = jax.lax.dot_general(
                h.astype(jnp.bfloat16), w2b, (((1,), (0,)), ((), ())),
                preferred_element_type=jnp.float32)

            @pl.when(f == 0)
            def _():
                y_ref[sl, :] = contrib

            @pl.when(f != 0)
            def _():
                y_ref[sl, :] = y_ref[sl, :] + contrib


def _merge_body(g0_ref, g1_ref, s0_ref, s1_ref, out_ref):
    s0 = s0_ref[...]
    s1 = s1_ref[...]
    out_ref[...] = (
        jnp.where(s0 != 0.0, g0_ref[...] * s0, 0.0)
        + jnp.where(s1 != 0.0, g1_ref[...] * s1, 0.0))


def kernel(hidden_states, w_router, w1, w2):
    T, D = hidden_states.shape
    E = w_router.shape[1]
    F = w1.shape[2]
    K = 2
    CAP = int(T * K / E * 1.25)
    TRASH = E * CAP
    NPAD = 8
    NBUF = TRASH + NPAD

    logits = hidden_states @ w_router

    dest, scale, aux, kc = pl.pallas_call(
        functools.partial(_router_body, T=T, E=E, K=K, CAP=CAP, TRASH=TRASH),
        out_shape=(
            jax.ShapeDtypeStruct((T, K), jnp.int32),
            jax.ShapeDtypeStruct((T, K), jnp.float32),
            jax.ShapeDtypeStruct((1, 1), jnp.float32),
            jax.ShapeDtypeStruct((1, E), jnp.int32),
        ),
    )(logits)

    TPW = T // _NW
    CH = min(TPW, 32)
    d0 = dest[:, 0].reshape(_NW, TPW // CH, CH)
    d1 = dest[:, 1].reshape(_NW, TPW // CH, CH)

    mesh = plsc.VectorSubcoreMesh(core_axis_name="c", subcore_axis_name="s")

    dispatch = functools.partial(
        pl.kernel,
        mesh=mesh,
        out_type=jax.ShapeDtypeStruct((NBUF, D), jnp.float32),
        scratch_types=[
            pltpu.VMEM((CH, D), jnp.float32),
            pltpu.VMEM((TPW // CH, CH), jnp.int32),
            pltpu.VMEM((TPW // CH, CH), jnp.int32),
            pltpu.SemaphoreType.DMA,
        ],
    )(functools.partial(_dispatch_body, tpw=TPW, ch=CH))
    buf = dispatch(hidden_states, d0, d1)

    FB = min(F, 1024)
    NF = F // FB
    RBF = 128
    y = pl.pallas_call(
        functools.partial(_ffn_body, rbf=RBF, cap=CAP),
        grid_spec=pltpu.PrefetchScalarGridSpec(
            num_scalar_prefetch=1,
            grid=(E, NF),
            in_specs=[
                pl.BlockSpec((CAP, D), lambda e, f, kc: (e, 0)),
                pl.BlockSpec((1, D, FB), lambda e, f, kc: (e, 0, f)),
                pl.BlockSpec((1, FB, D), lambda e, f, kc: (e, f, 0)),
            ],
            out_specs=pl.BlockSpec((CAP, D), lambda e, f, kc: (e, 0)),
        ),
        out_shape=jax.ShapeDtypeStruct((NBUF, D), jnp.float32),
    )(kc.reshape(E), buf, w1, w2)

    combine = functools.partial(
        pl.kernel,
        mesh=mesh,
        out_type=(
            jax.ShapeDtypeStruct((T, D), jnp.float32),
            jax.ShapeDtypeStruct((T, D), jnp.float32),
        ),
        scratch_types=[
            pltpu.VMEM((CH, D), jnp.float32),
            pltpu.VMEM((TPW // CH, CH), jnp.int32),
            pltpu.VMEM((TPW // CH, CH), jnp.int32),
            pltpu.SemaphoreType.DMA,
        ],
    )(functools.partial(_combine_body, tpw=TPW, ch=CH))
    g0, g1 = combine(y, d0, d1)

    RB = 256
    out = pl.pallas_call(
        _merge_body,
        grid=(T // RB,),
        in_specs=[
            pl.BlockSpec((RB, D), lambda i: (i, 0)),
            pl.BlockSpec((RB, D), lambda i: (i, 0)),
            pl.BlockSpec((RB, 1), lambda i: (i, 0)),
            pl.BlockSpec((RB, 1), lambda i: (i, 0)),
        ],
        out_specs=pl.BlockSpec((RB, D), lambda i: (i, 0)),
        out_shape=jax.ShapeDtypeStruct((T, D), jnp.float32),
    )(g0, g1, scale[:, 0:1], scale[:, 1:2])

    return out, aux.reshape(())


# speedup vs baseline: 1.2566x; 1.2566x over previous
import functools

import jax
import jax.numpy as jnp
from jax import lax
from jax.experimental import pallas as pl
from jax.experimental.pallas import tpu as pltpu
from jax.experimental.pallas import tpu_sc as plsc

_NC = 2
_NS = 16
_NW = _NC * _NS


def _router_body(logits_ref, dest_ref, scale_ref, aux_ref, kc_ref, *, T, E, K, CAP, TRASH):
    logits = logits_ref[...]
    iota_e = jax.lax.broadcasted_iota(jnp.int32, (T, E), 1)
    m1 = jnp.max(logits, axis=1, keepdims=True)
    a1 = jnp.min(jnp.where(logits == m1, iota_e, E), axis=1, keepdims=True)
    neg = jnp.float32(-jnp.inf)
    masked = jnp.where(iota_e == a1, neg, logits)
    m2 = jnp.max(masked, axis=1, keepdims=True)
    a2 = jnp.min(jnp.where(masked == m2, iota_e, E), axis=1, keepdims=True)
    z = jnp.exp(m2 - m1)
    g1 = 1.0 / (1.0 + z)
    g2 = z / (1.0 + z)
    p = jnp.exp(logits - m1)
    probs = p / jnp.sum(p, axis=1, keepdims=True)
    imp = jnp.mean(probs, axis=0, keepdims=True)
    oh1 = (iota_e == a1).astype(jnp.float32)
    oh2 = (iota_e == a2).astype(jnp.float32)
    c = oh1 + oh2
    counts = jnp.sum(c, axis=0, keepdims=True)
    aux_ref[...] = jnp.reshape(E * jnp.sum(imp * counts) / (T * K), (1, 1))
    kc_ref[...] = jnp.minimum(counts, CAP).astype(jnp.int32)
    s = c
    sh = 1
    while sh < T:
        s = s + jnp.concatenate(
            [jnp.zeros((sh, E), jnp.float32), s[: T - sh]], axis=0)
        sh *= 2
    cex = s - c
    pos1 = jnp.sum(cex * oh1, axis=1, keepdims=True).astype(jnp.int32)
    pos2 = jnp.sum(cex * oh2, axis=1, keepdims=True).astype(jnp.int32)
    keep1 = pos1 < CAP
    keep2 = pos2 < CAP
    d1 = jnp.where(keep1, a1 * CAP + jnp.minimum(pos1, CAP - 1), TRASH)
    d2 = jnp.where(keep2, a2 * CAP + jnp.minimum(pos2, CAP - 1), TRASH)
    dest_ref[...] = jnp.concatenate([d1, d2], axis=1)
    scale_ref[...] = jnp.concatenate(
        [jnp.where(keep1, g1, 0.0), jnp.where(keep2, g2, 0.0)], axis=1)


def _dispatch_body(hs_hbm, d0_hbm, d1_hbm, buf_hbm, rows_v, idx0_v, idx1_v, sem,
                   *, tpw, ch):
    wid = lax.axis_index("s") * _NC + lax.axis_index("c")
    base = wid * tpw
    pltpu.sync_copy(d0_hbm.at[wid], idx0_v)
    pltpu.sync_copy(d1_hbm.at[wid], idx1_v)
    for j in range(tpw // ch):
        pltpu.sync_copy(hs_hbm.at[pl.ds(base + j * ch, ch)], rows_v)
        pltpu.async_copy(rows_v, buf_hbm.at[idx0_v.at[j]], sem).wait()
        pltpu.async_copy(rows_v, buf_hbm.at[idx1_v.at[j]], sem).wait()


def _combine_body(y_hbm, d0_hbm, d1_hbm, g0_hbm, g1_hbm, rows_v, idx0_v, idx1_v,
                  sem, *, tpw, ch):
    wid = lax.axis_index("s") * _NC + lax.axis_index("c")
    base = wid * tpw
    pltpu.sync_copy(d0_hbm.at[wid], idx0_v)
    pltpu.sync_copy(d1_hbm.at[wid], idx1_v)
    for j in range(tpw // ch):
        pltpu.async_copy(y_hbm.at[idx0_v.at[j]], rows_v, sem).wait()
        pltpu.sync_copy(rows_v, g0_hbm.at[pl.ds(base + j * ch, ch)])
        pltpu.async_copy(y_hbm.at[idx1_v.at[j]], rows_v, sem).wait()
        pltpu.sync_copy(rows_v, g1_hbm.at[pl.ds(base + j * ch, ch)])


def _ffn_body(kc_ref, buf_ref, w1_ref, w2_ref, y_ref):
    f = pl.program_id(1)
    h = jax.nn.gelu(jax.lax.dot_general(
        buf_ref[...], w1_ref[0], (((1,), (0,)), ((), ())),
        preferred_element_type=jnp.float32))
    contrib = jax.lax.dot_general(
        h, w2_ref[0], (((1,), (0,)), ((), ())),
        preferred_element_type=jnp.float32)

    @pl.when(f == 0)
    def _():
        y_ref[...] = contrib

    @pl.when(f != 0)
    def _():
        y_ref[...] = y_ref[...] + contrib


def _merge_body(g0_ref, g1_ref, s0_ref, s1_ref, out_ref):
    s0 = s0_ref[...]
    s1 = s1_ref[...]
    out_ref[...] = (
        jnp.where(s0 != 0.0, g0_ref[...] * s0, 0.0)
        + jnp.where(s1 != 0.0, g1_ref[...] * s1, 0.0))


def kernel(hidden_states, w_router, w1, w2):
    T, D = hidden_states.shape
    E = w_router.shape[1]
    F = w1.shape[2]
    K = 2
    CAP = int(T * K / E * 1.25)
    TRASH = E * CAP
    NPAD = 8
    NBUF = TRASH + NPAD

    logits = hidden_states @ w_router

    dest, scale, aux, kc = pl.pallas_call(
        functools.partial(_router_body, T=T, E=E, K=K, CAP=CAP, TRASH=TRASH),
        out_shape=(
            jax.ShapeDtypeStruct((T, K), jnp.int32),
            jax.ShapeDtypeStruct((T, K), jnp.float32),
            jax.ShapeDtypeStruct((1, 1), jnp.float32),
            jax.ShapeDtypeStruct((1, E), jnp.int32),
        ),
    )(logits)

    TPW = T // _NW
    CH = min(TPW, 32)
    d0 = dest[:, 0].reshape(_NW, TPW // CH, CH)
    d1 = dest[:, 1].reshape(_NW, TPW // CH, CH)

    mesh = plsc.VectorSubcoreMesh(core_axis_name="c", subcore_axis_name="s")

    dispatch = functools.partial(
        pl.kernel,
        mesh=mesh,
        out_type=jax.ShapeDtypeStruct((NBUF, D), jnp.float32),
        scratch_types=[
            pltpu.VMEM((CH, D), jnp.float32),
            pltpu.VMEM((TPW // CH, CH), jnp.int32),
            pltpu.VMEM((TPW // CH, CH), jnp.int32),
            pltpu.SemaphoreType.DMA,
        ],
    )(functools.partial(_dispatch_body, tpw=TPW, ch=CH))
    buf = dispatch(hidden_states, d0, d1)

    FB = min(F, 1024)
    NF = F // FB
    y = pl.pallas_call(
        _ffn_body,
        grid_spec=pltpu.PrefetchScalarGridSpec(
            num_scalar_prefetch=1,
            grid=(E, NF),
            in_specs=[
                pl.BlockSpec((CAP, D), lambda e, f, kc: (e, 0)),
                pl.BlockSpec((1, D, FB), lambda e, f, kc: (e, 0, f)),
                pl.BlockSpec((1, FB, D), lambda e, f, kc: (e, f, 0)),
            ],
            out_specs=pl.BlockSpec((CAP, D), lambda e, f, kc: (e, 0)),
        ),
        out_shape=jax.ShapeDtypeStruct((NBUF, D), jnp.float32),
    )(kc.reshape(E), buf, w1, w2)

    combine = functools.partial(
        pl.kernel,
        mesh=mesh,
        out_type=(
            jax.ShapeDtypeStruct((T, D), jnp.float32),
            jax.ShapeDtypeStruct((T, D), jnp.float32),
        ),
        scratch_types=[
            pltpu.VMEM((CH, D), jnp.float32),
            pltpu.VMEM((TPW // CH, CH), jnp.int32),
            pltpu.VMEM((TPW // CH, CH), jnp.int32),
            pltpu.SemaphoreType.DMA,
        ],
    )(functools.partial(_combine_body, tpw=TPW, ch=CH))
    g0, g1 = combine(y, d0, d1)

    RB = 256
    out = pl.pallas_call(
        _merge_body,
        grid=(T // RB,),
        in_specs=[
            pl.BlockSpec((RB, D), lambda i: (i, 0)),
            pl.BlockSpec((RB, D), lambda i: (i, 0)),
            pl.BlockSpec((RB, 1), lambda i: (i, 0)),
            pl.BlockSpec((RB, 1), lambda i: (i, 0)),
        ],
        out_specs=pl.BlockSpec((RB, D), lambda i: (i, 0)),
        out_shape=jax.ShapeDtypeStruct((T, D), jnp.float32),
    )(g0, g1, scale[:, 0:1], scale[:, 1:2])

    return out, aux.reshape(())


# speedup vs baseline: 1.3053x; 1.0387x over previous
import functools

import jax
import jax.numpy as jnp
from jax import lax
from jax.experimental import pallas as pl
from jax.experimental.pallas import tpu as pltpu
from jax.experimental.pallas import tpu_sc as plsc

_NC = 2
_NS = 16
_NW = _NC * _NS


def _router_body(logits_ref, dest_ref, s0r_ref, s1r_ref, aux_ref, *, T, E, K, CAP, TRASH):
    logits = logits_ref[...]
    iota_e = jax.lax.broadcasted_iota(jnp.int32, (T, E), 1)
    m1 = jnp.max(logits, axis=1, keepdims=True)
    a1 = jnp.min(jnp.where(logits == m1, iota_e, E), axis=1, keepdims=True)
    neg = jnp.float32(-jnp.inf)
    masked = jnp.where(iota_e == a1, neg, logits)
    m2 = jnp.max(masked, axis=1, keepdims=True)
    a2 = jnp.min(jnp.where(masked == m2, iota_e, E), axis=1, keepdims=True)
    z = jnp.exp(m2 - m1)
    g1 = 1.0 / (1.0 + z)
    g2 = z / (1.0 + z)
    p = jnp.exp(logits - m1)
    probs = p / jnp.sum(p, axis=1, keepdims=True)
    imp = jnp.mean(probs, axis=0, keepdims=True)
    oh1 = (iota_e == a1).astype(jnp.float32)
    oh2 = (iota_e == a2).astype(jnp.float32)
    c = oh1 + oh2
    counts = jnp.sum(c, axis=0, keepdims=True)
    aux_ref[...] = jnp.reshape(E * jnp.sum(imp * counts) / (T * K), (1, 1))
    s = c
    sh = 1
    while sh < T:
        s = s + jnp.concatenate(
            [jnp.zeros((sh, E), jnp.float32), s[: T - sh]], axis=0)
        sh *= 2
    cex = s - c
    pos1 = jnp.sum(cex * oh1, axis=1, keepdims=True).astype(jnp.int32)
    pos2 = jnp.sum(cex * oh2, axis=1, keepdims=True).astype(jnp.int32)
    keep1 = pos1 < CAP
    keep2 = pos2 < CAP
    d1 = jnp.where(keep1, a1 * CAP + jnp.minimum(pos1, CAP - 1), TRASH)
    d2 = jnp.where(keep2, a2 * CAP + jnp.minimum(pos2, CAP - 1), TRASH)
    dest_ref[...] = jnp.concatenate([d1, d2], axis=1)
    s1 = jnp.where(keep1, g1, 0.0)
    s2 = jnp.where(keep2, g2, 0.0)
    s0r_ref[...] = jnp.broadcast_to(s1, (T, 16))
    s1r_ref[...] = jnp.broadcast_to(s2, (T, 16))


def _dispatch_body(hs_hbm, d0_hbm, d1_hbm, buf_hbm, rows_v, idx0_v, idx1_v, sem,
                   *, tpw, ch):
    wid = lax.axis_index("s") * _NC + lax.axis_index("c")
    base = wid * tpw
    pltpu.sync_copy(d0_hbm.at[wid], idx0_v)
    pltpu.sync_copy(d1_hbm.at[wid], idx1_v)
    for j in range(tpw // ch):
        pltpu.sync_copy(hs_hbm.at[pl.ds(base + j * ch, ch)], rows_v)
        pltpu.async_copy(rows_v, buf_hbm.at[idx0_v.at[j]], sem).wait()
        pltpu.async_copy(rows_v, buf_hbm.at[idx1_v.at[j]], sem).wait()


def _combine_body(y_hbm, d0_hbm, d1_hbm, s0_hbm, s1_hbm, out_hbm,
                  rows0_v, rows1_v, out_v, idx0_v, idx1_v, s0_v, s1_v,
                  sem0, sem1, *, tpw, ch, d):
    wid = lax.axis_index("s") * _NC + lax.axis_index("c")
    base = wid * tpw
    pltpu.sync_copy(d0_hbm.at[wid], idx0_v)
    pltpu.sync_copy(d1_hbm.at[wid], idx1_v)
    pltpu.sync_copy(s0_hbm.at[wid], s0_v)
    pltpu.sync_copy(s1_hbm.at[wid], s1_v)
    nvec = d // 16
    for j in range(tpw // ch):
        cp0 = pltpu.async_copy(y_hbm.at[idx0_v.at[j]], rows0_v, sem0)
        cp1 = pltpu.async_copy(y_hbm.at[idx1_v.at[j]], rows1_v, sem1)
        cp0.wait()
        cp1.wait()

        def row_fn(i, _):
            s0v = s0_v[j, pl.ds(i * 16, 16)]
            s1v = s1_v[j, pl.ds(i * 16, 16)]
            m0 = s0v != 0.0
            m1 = s1v != 0.0
            for kch in range(nvec):
                sl = pl.ds(kch * 16, 16)
                r0 = rows0_v[i, sl]
                r1 = rows1_v[i, sl]
                out_v[i, sl] = (jnp.where(m0, r0 * s0v, 0.0)
                                + jnp.where(m1, r1 * s1v, 0.0))
            return 0

        lax.fori_loop(0, ch, row_fn, 0)
        pltpu.sync_copy(out_v, out_hbm.at[pl.ds(base + j * ch, ch)])


def _ffn_body(buf_ref, w1_ref, w2_ref, y_ref):
    f = pl.program_id(1)
    h = jax.nn.gelu(jax.lax.dot_general(
        buf_ref[...], w1_ref[0], (((1,), (0,)), ((), ())),
        preferred_element_type=jnp.float32))
    contrib = jax.lax.dot_general(
        h, w2_ref[0], (((1,), (0,)), ((), ())),
        preferred_element_type=jnp.float32)

    @pl.when(f == 0)
    def _():
        y_ref[...] = contrib

    @pl.when(f != 0)
    def _():
        y_ref[...] = y_ref[...] + contrib


def kernel(hidden_states, w_router, w1, w2):
    T, D = hidden_states.shape
    E = w_router.shape[1]
    F = w1.shape[2]
    K = 2
    CAP = int(T * K / E * 1.25)
    TRASH = E * CAP
    NPAD = 8
    NBUF = TRASH + NPAD

    logits = hidden_states @ w_router

    dest, s0r, s1r, aux = pl.pallas_call(
        functools.partial(_router_body, T=T, E=E, K=K, CAP=CAP, TRASH=TRASH),
        out_shape=(
            jax.ShapeDtypeStruct((T, K), jnp.int32),
            jax.ShapeDtypeStruct((T, 16), jnp.float32),
            jax.ShapeDtypeStruct((T, 16), jnp.float32),
            jax.ShapeDtypeStruct((1, 1), jnp.float32),
        ),
    )(logits)

    TPW = T // _NW
    CH = min(TPW, 32)
    NCH = TPW // CH
    d0 = dest[:, 0].reshape(_NW, NCH, CH)
    d1 = dest[:, 1].reshape(_NW, NCH, CH)
    s0w = s0r.reshape(_NW, NCH, CH * 16)
    s1w = s1r.reshape(_NW, NCH, CH * 16)

    mesh = plsc.VectorSubcoreMesh(core_axis_name="c", subcore_axis_name="s")

    dispatch = functools.partial(
        pl.kernel,
        mesh=mesh,
        out_type=jax.ShapeDtypeStruct((NBUF, D), jnp.float32),
        scratch_types=[
            pltpu.VMEM((CH, D), jnp.float32),
            pltpu.VMEM((TPW // CH, CH), jnp.int32),
            pltpu.VMEM((TPW // CH, CH), jnp.int32),
            pltpu.SemaphoreType.DMA,
        ],
    )(functools.partial(_dispatch_body, tpw=TPW, ch=CH))
    buf = dispatch(hidden_states, d0, d1)

    FB = min(F, 1024)
    NF = F // FB
    y = pl.pallas_call(
        _ffn_body,
        grid=(E, NF),
        in_specs=[
            pl.BlockSpec((CAP, D), lambda e, f: (e, 0)),
            pl.BlockSpec((1, D, FB), lambda e, f: (e, 0, f)),
            pl.BlockSpec((1, FB, D), lambda e, f: (e, f, 0)),
        ],
        out_specs=pl.BlockSpec((CAP, D), lambda e, f: (e, 0)),
        out_shape=jax.ShapeDtypeStruct((NBUF, D), jnp.float32),
    )(buf, w1, w2)

    combine = functools.partial(
        pl.kernel,
        mesh=mesh,
        out_type=jax.ShapeDtypeStruct((T, D), jnp.float32),
        scratch_types=[
            pltpu.VMEM((CH, D), jnp.float32),
            pltpu.VMEM((CH, D), jnp.float32),
            pltpu.VMEM((CH, D), jnp.float32),
            pltpu.VMEM((NCH, CH), jnp.int32),
            pltpu.VMEM((NCH, CH), jnp.int32),
            pltpu.VMEM((NCH, CH * 16), jnp.float32),
            pltpu.VMEM((NCH, CH * 16), jnp.float32),
            pltpu.SemaphoreType.DMA,
            pltpu.SemaphoreType.DMA,
        ],
    )(functools.partial(_combine_body, tpw=TPW, ch=CH, d=D))
    out = combine(y, d0, d1, s0w, s1w)

    return out, aux.reshape(())


# speedup vs baseline: 1.3720x; 1.0511x over previous
import functools

import jax
import jax.numpy as jnp
from jax import lax
from jax.experimental import pallas as pl
from jax.experimental.pallas import tpu as pltpu
from jax.experimental.pallas import tpu_sc as plsc

_NC = 2
_NS = 16
_NW = _NC * _NS


def _router_body(logits_ref, d0_ref, d1_ref, s0r_ref, s1r_ref, aux_ref, *, T, E, K, CAP, TRASH):
    logits = logits_ref[...]
    iota_e = jax.lax.broadcasted_iota(jnp.int32, (T, E), 1)
    m1 = jnp.max(logits, axis=1, keepdims=True)
    a1 = jnp.min(jnp.where(logits == m1, iota_e, E), axis=1, keepdims=True)
    neg = jnp.float32(-jnp.inf)
    masked = jnp.where(iota_e == a1, neg, logits)
    m2 = jnp.max(masked, axis=1, keepdims=True)
    a2 = jnp.min(jnp.where(masked == m2, iota_e, E), axis=1, keepdims=True)
    z = jnp.exp(m2 - m1)
    g1 = 1.0 / (1.0 + z)
    g2 = z / (1.0 + z)
    p = jnp.exp(logits - m1)
    probs = p / jnp.sum(p, axis=1, keepdims=True)
    imp = jnp.mean(probs, axis=0, keepdims=True)
    oh1 = (iota_e == a1).astype(jnp.float32)
    oh2 = (iota_e == a2).astype(jnp.float32)
    c = oh1 + oh2
    counts = jnp.sum(c, axis=0, keepdims=True)
    aux_ref[...] = jnp.reshape(E * jnp.sum(imp * counts) / (T * K), (1, 1))
    s = c
    sh = 1
    while sh < T:
        s = s + jnp.concatenate(
            [jnp.zeros((sh, E), jnp.float32), s[: T - sh]], axis=0)
        sh *= 2
    cex = s - c
    pos1 = jnp.sum(cex * oh1, axis=1, keepdims=True).astype(jnp.int32)
    pos2 = jnp.sum(cex * oh2, axis=1, keepdims=True).astype(jnp.int32)
    keep1 = pos1 < CAP
    keep2 = pos2 < CAP
    d1 = jnp.where(keep1, a1 * CAP + jnp.minimum(pos1, CAP - 1), TRASH)
    d2 = jnp.where(keep2, a2 * CAP + jnp.minimum(pos2, CAP - 1), TRASH)
    d0_ref[...] = d1
    d1_ref[...] = d2
    s1 = jnp.where(keep1, g1, 0.0)
    s2 = jnp.where(keep2, g2, 0.0)
    s0r_ref[...] = jnp.broadcast_to(s1, (T, 16))
    s1r_ref[...] = jnp.broadcast_to(s2, (T, 16))


def _dispatch_body(hs_hbm, d0_hbm, d1_hbm, buf_hbm, rows_v, idx0_v, idx1_v, sem,
                   *, tpw, ch):
    wid = lax.axis_index("s") * _NC + lax.axis_index("c")
    base = wid * tpw
    pltpu.sync_copy(d0_hbm.at[wid], idx0_v)
    pltpu.sync_copy(d1_hbm.at[wid], idx1_v)
    for j in range(tpw // ch):
        pltpu.sync_copy(hs_hbm.at[pl.ds(base + j * ch, ch)], rows_v)
        pltpu.async_copy(rows_v, buf_hbm.at[idx0_v.at[j]], sem).wait()
        pltpu.async_copy(rows_v, buf_hbm.at[idx1_v.at[j]], sem).wait()


def _combine_body(y_hbm, d0_hbm, d1_hbm, s0_hbm, s1_hbm, out_hbm,
                  rows0_v, rows1_v, out_v, idx0_v, idx1_v, s0_v, s1_v,
                  sem0, sem1, *, tpw, ch, d):
    wid = lax.axis_index("s") * _NC + lax.axis_index("c")
    base = wid * tpw
    pltpu.sync_copy(d0_hbm.at[wid], idx0_v)
    pltpu.sync_copy(d1_hbm.at[wid], idx1_v)
    pltpu.sync_copy(s0_hbm.at[wid], s0_v)
    pltpu.sync_copy(s1_hbm.at[wid], s1_v)
    nvec = d // 16
    for j in range(tpw // ch):
        cp0 = pltpu.async_copy(y_hbm.at[idx0_v.at[j]], rows0_v, sem0)
        cp1 = pltpu.async_copy(y_hbm.at[idx1_v.at[j]], rows1_v, sem1)
        cp0.wait()
        cp1.wait()

        def row_fn(i, _):
            s0v = s0_v[j, pl.ds(i * 16, 16)]
            s1v = s1_v[j, pl.ds(i * 16, 16)]
            m0 = s0v != 0.0
            m1 = s1v != 0.0
            for kch in range(nvec):
                sl = pl.ds(kch * 16, 16)
                r0 = rows0_v[i, sl]
                r1 = rows1_v[i, sl]
                out_v[i, sl] = (jnp.where(m0, r0 * s0v, 0.0)
                                + jnp.where(m1, r1 * s1v, 0.0))
            return 0

        lax.fori_loop(0, ch, row_fn, 0)
        pltpu.sync_copy(out_v, out_hbm.at[pl.ds(base + j * ch, ch)])


def _ffn_body(buf_ref, w1_ref, w2_ref, y_ref):
    f = pl.program_id(1)
    h = jax.nn.gelu(jax.lax.dot_general(
        buf_ref[...], w1_ref[0], (((1,), (0,)), ((), ())),
        preferred_element_type=jnp.float32))
    contrib = jax.lax.dot_general(
        h, w2_ref[0], (((1,), (0,)), ((), ())),
        preferred_element_type=jnp.float32)

    @pl.when(f == 0)
    def _():
        y_ref[...] = contrib

    @pl.when(f != 0)
    def _():
        y_ref[...] = y_ref[...] + contrib


def kernel(hidden_states, w_router, w1, w2):
    T, D = hidden_states.shape
    E = w_router.shape[1]
    F = w1.shape[2]
    K = 2
    CAP = int(T * K / E * 1.25)
    TRASH = E * CAP
    NPAD = 8
    NBUF = TRASH + NPAD

    logits = hidden_states @ w_router

    d0c, d1c, s0r, s1r, aux = pl.pallas_call(
        functools.partial(_router_body, T=T, E=E, K=K, CAP=CAP, TRASH=TRASH),
        out_shape=(
            jax.ShapeDtypeStruct((T, 1), jnp.int32),
            jax.ShapeDtypeStruct((T, 1), jnp.int32),
            jax.ShapeDtypeStruct((T, 16), jnp.float32),
            jax.ShapeDtypeStruct((T, 16), jnp.float32),
            jax.ShapeDtypeStruct((1, 1), jnp.float32),
        ),
    )(logits)

    TPW = T // _NW
    CH = min(TPW, 32)
    NCH = TPW // CH
    d0 = d0c.reshape(_NW, NCH, CH)
    d1 = d1c.reshape(_NW, NCH, CH)
    s0w = s0r.reshape(_NW, NCH, CH * 16)
    s1w = s1r.reshape(_NW, NCH, CH * 16)

    mesh = plsc.VectorSubcoreMesh(core_axis_name="c", subcore_axis_name="s")

    dispatch = functools.partial(
        pl.kernel,
        mesh=mesh,
        out_type=jax.ShapeDtypeStruct((NBUF, D), jnp.float32),
        scratch_types=[
            pltpu.VMEM((CH, D), jnp.float32),
            pltpu.VMEM((TPW // CH, CH), jnp.int32),
            pltpu.VMEM((TPW // CH, CH), jnp.int32),
            pltpu.SemaphoreType.DMA,
        ],
    )(functools.partial(_dispatch_body, tpw=TPW, ch=CH))
    buf = dispatch(hidden_states, d0, d1)

    FB = min(F, 2048)
    NF = F // FB
    y = pl.pallas_call(
        _ffn_body,
        grid=(E, NF),
        in_specs=[
            pl.BlockSpec((CAP, D), lambda e, f: (e, 0)),
            pl.BlockSpec((1, D, FB), lambda e, f: (e, 0, f)),
            pl.BlockSpec((1, FB, D), lambda e, f: (e, f, 0)),
        ],
        out_specs=pl.BlockSpec((CAP, D), lambda e, f: (e, 0)),
        out_shape=jax.ShapeDtypeStruct((NBUF, D), jnp.float32),
    )(buf, w1, w2)

    combine = functools.partial(
        pl.kernel,
        mesh=mesh,
        out_type=jax.ShapeDtypeStruct((T, D), jnp.float32),
        scratch_types=[
            pltpu.VMEM((CH, D), jnp.float32),
            pltpu.VMEM((CH, D), jnp.float32),
            pltpu.VMEM((CH, D), jnp.float32),
            pltpu.VMEM((NCH, CH), jnp.int32),
            pltpu.VMEM((NCH, CH), jnp.int32),
            pltpu.VMEM((NCH, CH * 16), jnp.float32),
            pltpu.VMEM((NCH, CH * 16), jnp.float32),
            pltpu.SemaphoreType.DMA,
            pltpu.SemaphoreType.DMA,
        ],
    )(functools.partial(_combine_body, tpw=TPW, ch=CH, d=D))
    out = combine(y, d0, d1, s0w, s1w)

    return out, aux.reshape(())


# speedup vs baseline: 1.3734x; 1.0010x over previous
import functools

import jax
import jax.numpy as jnp
from jax import lax
from jax.experimental import pallas as pl
from jax.experimental.pallas import tpu as pltpu
from jax.experimental.pallas import tpu_sc as plsc

_NC = 2
_NS = 16
_NW = _NC * _NS


def _router_body(logits_ref, d0_ref, d1_ref, s0r_ref, s1r_ref, aux_ref, *, T, E, K, CAP, TRASH):
    logits = logits_ref[...]
    iota_e = jax.lax.broadcasted_iota(jnp.int32, (T, E), 1)
    m1 = jnp.max(logits, axis=1, keepdims=True)
    a1 = jnp.min(jnp.where(logits == m1, iota_e, E), axis=1, keepdims=True)
    neg = jnp.float32(-jnp.inf)
    masked = jnp.where(iota_e == a1, neg, logits)
    m2 = jnp.max(masked, axis=1, keepdims=True)
    a2 = jnp.min(jnp.where(masked == m2, iota_e, E), axis=1, keepdims=True)
    z = jnp.exp(m2 - m1)
    g1 = 1.0 / (1.0 + z)
    g2 = z / (1.0 + z)
    p = jnp.exp(logits - m1)
    probs = p / jnp.sum(p, axis=1, keepdims=True)
    imp = jnp.mean(probs, axis=0, keepdims=True)
    oh1 = (iota_e == a1).astype(jnp.float32)
    oh2 = (iota_e == a2).astype(jnp.float32)
    c = oh1 + oh2
    counts = jnp.sum(c, axis=0, keepdims=True)
    aux_ref[...] = jnp.reshape(E * jnp.sum(imp * counts) / (T * K), (1, 1))
    s = c
    sh = 1
    while sh < T:
        s = s + jnp.concatenate(
            [jnp.zeros((sh, E), jnp.float32), s[: T - sh]], axis=0)
        sh *= 2
    cex = s - c
    pos1 = jnp.sum(cex * oh1, axis=1, keepdims=True).astype(jnp.int32)
    pos2 = jnp.sum(cex * oh2, axis=1, keepdims=True).astype(jnp.int32)
    keep1 = pos1 < CAP
    keep2 = pos2 < CAP
    d1 = jnp.where(keep1, a1 * CAP + jnp.minimum(pos1, CAP - 1), TRASH)
    d2 = jnp.where(keep2, a2 * CAP + jnp.minimum(pos2, CAP - 1), TRASH)
    d0_ref[...] = d1
    d1_ref[...] = d2
    s1 = jnp.where(keep1, g1, 0.0)
    s2 = jnp.where(keep2, g2, 0.0)
    s0r_ref[...] = jnp.broadcast_to(s1, (T, 16))
    s1r_ref[...] = jnp.broadcast_to(s2, (T, 16))


def _dispatch_body(hs_hbm, d0_hbm, d1_hbm, buf_hbm, rows_a, rows_b,
                   idx0_v, idx1_v, sem_g, sem_s0, sem_s1, *, tpw, ch):
    wid = lax.axis_index("s") * _NC + lax.axis_index("c")
    base = wid * tpw
    pltpu.sync_copy(d0_hbm.at[wid], idx0_v)
    pltpu.sync_copy(d1_hbm.at[wid], idx1_v)
    pending = []
    for j in range(tpw // ch):
        rv = rows_a if j % 2 == 0 else rows_b
        pltpu.async_copy(hs_hbm.at[pl.ds(base + j * ch, ch)], rv, sem_g).wait()
        for h in pending:
            h.wait()
        pending = [
            pltpu.async_copy(rv, buf_hbm.at[idx0_v.at[j]], sem_s0),
            pltpu.async_copy(rv, buf_hbm.at[idx1_v.at[j]], sem_s1),
        ]
    for h in pending:
        h.wait()


def _combine_body(y_hbm, d0_hbm, d1_hbm, s0_hbm, s1_hbm, out_hbm,
                  rows0_v, rows1_v, out_v, idx0_v, idx1_v, s0_v, s1_v,
                  sem0, sem1, *, tpw, ch, d):
    wid = lax.axis_index("s") * _NC + lax.axis_index("c")
    base = wid * tpw
    pltpu.sync_copy(d0_hbm.at[wid], idx0_v)
    pltpu.sync_copy(d1_hbm.at[wid], idx1_v)
    pltpu.sync_copy(s0_hbm.at[wid], s0_v)
    pltpu.sync_copy(s1_hbm.at[wid], s1_v)
    nvec = d // 16
    for j in range(tpw // ch):
        cp0 = pltpu.async_copy(y_hbm.at[idx0_v.at[j]], rows0_v, sem0)
        cp1 = pltpu.async_copy(y_hbm.at[idx1_v.at[j]], rows1_v, sem1)
        cp0.wait()
        cp1.wait()

        def row_fn(i, _):
            s0v = s0_v[j, pl.ds(i * 16, 16)]
            s1v = s1_v[j, pl.ds(i * 16, 16)]
            m0 = s0v != 0.0
            m1 = s1v != 0.0
            for kch in range(nvec):
                sl = pl.ds(kch * 16, 16)
                r0 = rows0_v[i, sl]
                r1 = rows1_v[i, sl]
                out_v[i, sl] = (jnp.where(m0, r0 * s0v, 0.0)
                                + jnp.where(m1, r1 * s1v, 0.0))
            return 0

        lax.fori_loop(0, ch, row_fn, 0)
        pltpu.sync_copy(out_v, out_hbm.at[pl.ds(base + j * ch, ch)])


def _ffn_body(buf_ref, w1_ref, w2_ref, y_ref):
    f = pl.program_id(1)
    h = jax.nn.gelu(jax.lax.dot_general(
        buf_ref[...], w1_ref[0], (((1,), (0,)), ((), ())),
        preferred_element_type=jnp.float32))
    contrib = jax.lax.dot_general(
        h, w2_ref[0], (((1,), (0,)), ((), ())),
        preferred_element_type=jnp.float32)

    @pl.when(f == 0)
    def _():
        y_ref[...] = contrib

    @pl.when(f != 0)
    def _():
        y_ref[...] = y_ref[...] + contrib


def kernel(hidden_states, w_router, w1, w2):
    T, D = hidden_states.shape
    E = w_router.shape[1]
    F = w1.shape[2]
    K = 2
    CAP = int(T * K / E * 1.25)
    TRASH = E * CAP
    NPAD = 8
    NBUF = TRASH + NPAD

    logits = hidden_states @ w_router

    d0c, d1c, s0r, s1r, aux = pl.pallas_call(
        functools.partial(_router_body, T=T, E=E, K=K, CAP=CAP, TRASH=TRASH),
        out_shape=(
            jax.ShapeDtypeStruct((T, 1), jnp.int32),
            jax.ShapeDtypeStruct((T, 1), jnp.int32),
            jax.ShapeDtypeStruct((T, 16), jnp.float32),
            jax.ShapeDtypeStruct((T, 16), jnp.float32),
            jax.ShapeDtypeStruct((1, 1), jnp.float32),
        ),
    )(logits)

    TPW = T // _NW
    CH = min(TPW, 32)
    NCH = TPW // CH
    d0 = d0c.reshape(_NW, NCH, CH)
    d1 = d1c.reshape(_NW, NCH, CH)
    s0w = s0r.reshape(_NW, NCH, CH * 16)
    s1w = s1r.reshape(_NW, NCH, CH * 16)

    mesh = plsc.VectorSubcoreMesh(core_axis_name="c", subcore_axis_name="s")

    dispatch = functools.partial(
        pl.kernel,
        mesh=mesh,
        out_type=jax.ShapeDtypeStruct((NBUF, D), jnp.float32),
        scratch_types=[
            pltpu.VMEM((CH, D), jnp.float32),
            pltpu.VMEM((CH, D), jnp.float32),
            pltpu.VMEM((NCH, CH), jnp.int32),
            pltpu.VMEM((NCH, CH), jnp.int32),
            pltpu.SemaphoreType.DMA,
            pltpu.SemaphoreType.DMA,
            pltpu.SemaphoreType.DMA,
        ],
    )(functools.partial(_dispatch_body, tpw=TPW, ch=CH))
    buf = dispatch(hidden_states, d0, d1)

    FB = min(F, 2048)
    NF = F // FB
    y = pl.pallas_call(
        _ffn_body,
        grid=(E, NF),
        in_specs=[
            pl.BlockSpec((CAP, D), lambda e, f: (e, 0)),
            pl.BlockSpec((1, D, FB), lambda e, f: (e, 0, f)),
            pl.BlockSpec((1, FB, D), lambda e, f: (e, f, 0)),
        ],
        out_specs=pl.BlockSpec((CAP, D), lambda e, f: (e, 0)),
        out_shape=jax.ShapeDtypeStruct((NBUF, D), jnp.float32),
    )(buf, w1, w2)

    combine = functools.partial(
        pl.kernel,
        mesh=mesh,
        out_type=jax.ShapeDtypeStruct((T, D), jnp.float32),
        scratch_types=[
            pltpu.VMEM((CH, D), jnp.float32),
            pltpu.VMEM((CH, D), jnp.float32),
            pltpu.VMEM((CH, D), jnp.float32),
            pltpu.VMEM((NCH, CH), jnp.int32),
            pltpu.VMEM((NCH, CH), jnp.int32),
            pltpu.VMEM((NCH, CH * 16), jnp.float32),
            pltpu.VMEM((NCH, CH * 16), jnp.float32),
            pltpu.SemaphoreType.DMA,
            pltpu.SemaphoreType.DMA,
        ],
    )(functools.partial(_combine_body, tpw=TPW, ch=CH, d=D))
    out = combine(y, d0, d1, s0w, s1w)

    return out, aux.reshape(())
